# Initial kernel scaffold; baseline (speedup 1.0000x reference)
import jax
import jax.numpy as jnp
from jax.experimental import pallas as pl


def kernel(vdata, edata, connectivity, W, b):
    raise NotImplementedError("write your pallas kernel here")


# speedup vs baseline: 3.1978x; 3.1978x over previous
import functools

import jax
import jax.numpy as jnp
from jax import lax
from jax.experimental import pallas as pl
from jax.experimental.pallas import tpu as pltpu
from jax.experimental.pallas import tpu_sc as plsc

N_NODES = 10000
N_EDGES = 320000
D_FEAT = 128
D_EDGE = 16


_TC1_BLOCK = 1000


def _tc1_body(vd_ref, ws_ref, wr_ref, ps_ref, pr_ref):
    vd = vd_ref[...]
    ps_ref[...] = jnp.dot(vd, ws_ref[...], preferred_element_type=jnp.float32)
    pr_ref[...] = jnp.dot(vd, wr_ref[...], preferred_element_type=jnp.float32)


def _node_projections(vdata, ws_t, wr_t):
    grid = N_NODES // _TC1_BLOCK
    return pl.pallas_call(
        _tc1_body,
        grid=(grid,),
        in_specs=[
            pl.BlockSpec((_TC1_BLOCK, D_FEAT), lambda i: (i, 0)),
            pl.BlockSpec((D_FEAT, D_FEAT), lambda i: (0, 0)),
            pl.BlockSpec((D_FEAT, D_FEAT), lambda i: (0, 0)),
        ],
        out_specs=[
            pl.BlockSpec((_TC1_BLOCK, D_FEAT), lambda i: (i, 0)),
            pl.BlockSpec((_TC1_BLOCK, D_FEAT), lambda i: (i, 0)),
        ],
        out_shape=[
            jax.ShapeDtypeStruct((N_NODES, D_FEAT), jnp.float32),
            jax.ShapeDtypeStruct((N_NODES, D_FEAT), jnp.float32),
        ],
    )(vdata, ws_t, wr_t)


_C = 128
_NCHUNK = N_EDGES // _C
_NW = 32
_ITERS = (_NCHUNK + _NW - 1) // _NW


def _sc_gather_sum(senders, receivers, ps, pr):
    mesh = plsc.VectorSubcoreMesh(core_axis_name="c", subcore_axis_name="s")

    @functools.partial(
        pl.kernel,
        mesh=mesh,
        out_type=jax.ShapeDtypeStruct((N_EDGES, D_FEAT), jnp.float32),
        scratch_types=[
            pltpu.VMEM((_C,), jnp.int32),
            pltpu.VMEM((_C,), jnp.int32),
            pltpu.VMEM((_C, D_FEAT), jnp.float32),
            pltpu.VMEM((_C, D_FEAT), jnp.float32),
            pltpu.SemaphoreType.DMA,
            pltpu.SemaphoreType.DMA,
        ],
    )
    def k(sidx_hbm, ridx_hbm, ps_hbm, pr_hbm, out_hbm,
          sidx_v, ridx_v, rows_s, rows_r, sem_s, sem_r):
        wid = lax.axis_index("s") * 2 + lax.axis_index("c")

        def chunk_body(i, carry):
            cid = wid + i * _NW

            @pl.when(cid < _NCHUNK)
            def _():
                off = cid * _C
                pltpu.sync_copy(sidx_hbm.at[pl.ds(off, _C)], sidx_v)
                pltpu.sync_copy(ridx_hbm.at[pl.ds(off, _C)], ridx_v)
                cp_s = pltpu.async_copy(ps_hbm.at[sidx_v], rows_s, sem_s)
                cp_r = pltpu.async_copy(pr_hbm.at[ridx_v], rows_r, sem_r)
                cp_s.wait()
                cp_r.wait()

                def add_body(e, c):
                    for kk in range(D_FEAT // 16):
                        sl = pl.ds(kk * 16, 16)
                        rows_s[e, sl] = rows_s[e, sl] + rows_r[e, sl]
                    return c

                lax.fori_loop(0, _C, add_body, 0)
                pltpu.sync_copy(rows_s, out_hbm.at[pl.ds(off, _C)])

            return carry

        lax.fori_loop(0, _ITERS, chunk_body, 0)

    return k(senders, receivers, ps, pr)


_TC2_BLOCK = 4000


def _tc2_body(g_ref, ed_ref, we_ref, b_ref, out_ref):
    prod = jnp.dot(ed_ref[...], we_ref[...], preferred_element_type=jnp.float32)
    out_ref[...] = g_ref[...] + prod + b_ref[...]


def _edge_update(gathered, edata, we_t, b2d):
    grid = N_EDGES // _TC2_BLOCK
    return pl.pallas_call(
        _tc2_body,
        grid=(grid,),
        in_specs=[
            pl.BlockSpec((_TC2_BLOCK, D_FEAT), lambda i: (i, 0)),
            pl.BlockSpec((_TC2_BLOCK, D_EDGE), lambda i: (i, 0)),
            pl.BlockSpec((D_EDGE, D_FEAT), lambda i: (0, 0)),
            pl.BlockSpec((1, D_FEAT), lambda i: (0, 0)),
        ],
        out_specs=pl.BlockSpec((_TC2_BLOCK, D_FEAT), lambda i: (i, 0)),
        out_shape=jax.ShapeDtypeStruct((N_EDGES, D_FEAT), jnp.float32),
    )(gathered, edata, we_t, b2d)


def kernel(vdata, edata, connectivity, W, b):
    senders = connectivity[0].astype(jnp.int32)
    receivers = connectivity[1].astype(jnp.int32)
    we_t = W[:, :D_EDGE].T
    ws_t = W[:, D_EDGE:D_EDGE + D_FEAT].T
    wr_t = W[:, D_EDGE + D_FEAT:].T
    ps, pr = _node_projections(vdata, ws_t, wr_t)
    gathered = _sc_gather_sum(senders, receivers, ps, pr)
    return _edge_update(gathered, edata, we_t, b.reshape(1, D_FEAT))


# speedup vs baseline: 3.7071x; 1.1593x over previous
import functools

import jax
import jax.numpy as jnp
from jax import lax
from jax.experimental import pallas as pl
from jax.experimental.pallas import tpu as pltpu
from jax.experimental.pallas import tpu_sc as plsc

N_NODES = 10000
N_EDGES = 320000
D_FEAT = 128
D_EDGE = 16


_TC1_BLOCK = 1000


def _tc1_body(vd_ref, ws_ref, wr_ref, ps_ref, pr_ref):
    vd = vd_ref[...]
    ps_ref[...] = jnp.dot(vd, ws_ref[...], preferred_element_type=jnp.float32)
    pr_ref[...] = jnp.dot(vd, wr_ref[...], preferred_element_type=jnp.float32)


def _node_projections(vdata, ws_t, wr_t):
    grid = N_NODES // _TC1_BLOCK
    return pl.pallas_call(
        _tc1_body,
        grid=(grid,),
        in_specs=[
            pl.BlockSpec((_TC1_BLOCK, D_FEAT), lambda i: (i, 0)),
            pl.BlockSpec((D_FEAT, D_FEAT), lambda i: (0, 0)),
            pl.BlockSpec((D_FEAT, D_FEAT), lambda i: (0, 0)),
        ],
        out_specs=[
            pl.BlockSpec((_TC1_BLOCK, D_FEAT), lambda i: (i, 0)),
            pl.BlockSpec((_TC1_BLOCK, D_FEAT), lambda i: (i, 0)),
        ],
        out_shape=[
            jax.ShapeDtypeStruct((N_NODES, D_FEAT), jnp.float32),
            jax.ShapeDtypeStruct((N_NODES, D_FEAT), jnp.float32),
        ],
    )(vdata, ws_t, wr_t)


_NW = 32
_EPW = N_EDGES // _NW
_C = 128
_FULL = _EPW // _C
_TAIL = _EPW - _FULL * _C


def _sc_gather_sum(senders, receivers, ps, pr):
    mesh = plsc.VectorSubcoreMesh(core_axis_name="c", subcore_axis_name="s")

    @functools.partial(
        pl.kernel,
        mesh=mesh,
        out_type=jax.ShapeDtypeStruct((N_EDGES, D_FEAT), jnp.float32),
        scratch_types=[
            pltpu.VMEM((_EPW,), jnp.int32),
            pltpu.VMEM((_EPW,), jnp.int32),
            pltpu.VMEM((_C, D_FEAT), jnp.float32),
            pltpu.SemaphoreType.DMA,
        ],
    )
    def k(sidx_hbm, ridx_hbm, ps_hbm, pr_hbm, out_hbm,
          sidx_v, ridx_v, rows, sem):
        wid = lax.axis_index("s") * 2 + lax.axis_index("c")
        base = wid * _EPW
        pltpu.sync_copy(sidx_hbm.at[pl.ds(base, _EPW)], sidx_v)
        pltpu.sync_copy(ridx_hbm.at[pl.ds(base, _EPW)], ridx_v)

        def chunk_body(i, carry):
            off = i * _C
            pltpu.async_copy(
                ps_hbm.at[sidx_v.at[pl.ds(off, _C)]], rows, sem).wait()
            pltpu.async_copy(
                pr_hbm.at[ridx_v.at[pl.ds(off, _C)]], rows, sem,
                add=True).wait()
            pltpu.sync_copy(rows, out_hbm.at[pl.ds(base + off, _C)])
            return carry

        lax.fori_loop(0, _FULL, chunk_body, 0)

        toff = _FULL * _C
        rows_t = rows.at[pl.ds(0, _TAIL)]
        pltpu.async_copy(
            ps_hbm.at[sidx_v.at[pl.ds(toff, _TAIL)]], rows_t, sem).wait()
        pltpu.async_copy(
            pr_hbm.at[ridx_v.at[pl.ds(toff, _TAIL)]], rows_t, sem,
            add=True).wait()
        pltpu.sync_copy(rows_t, out_hbm.at[pl.ds(base + toff, _TAIL)])

    return k(senders, receivers, ps, pr)


_TC2_BLOCK = 4000


def _tc2_body(g_ref, ed_ref, we_ref, b_ref, out_ref):
    prod = jnp.dot(ed_ref[...], we_ref[...], preferred_element_type=jnp.float32)
    out_ref[...] = g_ref[...] + prod + b_ref[...]


def _edge_update(gathered, edata, we_t, b2d):
    grid = N_EDGES // _TC2_BLOCK
    return pl.pallas_call(
        _tc2_body,
        grid=(grid,),
        in_specs=[
            pl.BlockSpec((_TC2_BLOCK, D_FEAT), lambda i: (i, 0)),
            pl.BlockSpec((_TC2_BLOCK, D_EDGE), lambda i: (i, 0)),
            pl.BlockSpec((D_EDGE, D_FEAT), lambda i: (0, 0)),
            pl.BlockSpec((1, D_FEAT), lambda i: (0, 0)),
        ],
        out_specs=pl.BlockSpec((_TC2_BLOCK, D_FEAT), lambda i: (i, 0)),
        out_shape=jax.ShapeDtypeStruct((N_EDGES, D_FEAT), jnp.float32),
    )(gathered, edata, we_t, b2d)


def kernel(vdata, edata, connectivity, W, b):
    senders = connectivity[0].astype(jnp.int32)
    receivers = connectivity[1].astype(jnp.int32)
    we_t = W[:, :D_EDGE].T
    ws_t = W[:, D_EDGE:D_EDGE + D_FEAT].T
    wr_t = W[:, D_EDGE + D_FEAT:].T
    ps, pr = _node_projections(vdata, ws_t, wr_t)
    gathered = _sc_gather_sum(senders, receivers, ps, pr)
    return _edge_update(gathered, edata, we_t, b.reshape(1, D_FEAT))


# speedup vs baseline: 4.2256x; 1.1399x over previous
import functools

import jax
import jax.numpy as jnp
from jax import lax
from jax.experimental import pallas as pl
from jax.experimental.pallas import tpu as pltpu
from jax.experimental.pallas import tpu_sc as plsc

N_NODES = 10000
N_EDGES = 320000
D_FEAT = 128
D_EDGE = 16


_TC1_BLOCK = 1000


def _tc1_body(vd_ref, ws_ref, wr_ref, ps_ref, pr_ref):
    vd = vd_ref[...]
    ps_ref[...] = jnp.dot(vd, ws_ref[...], preferred_element_type=jnp.float32)
    pr_ref[...] = jnp.dot(vd, wr_ref[...], preferred_element_type=jnp.float32)


def _node_projections(vdata, ws_t, wr_t):
    grid = N_NODES // _TC1_BLOCK
    return pl.pallas_call(
        _tc1_body,
        grid=(grid,),
        in_specs=[
            pl.BlockSpec((_TC1_BLOCK, D_FEAT), lambda i: (i, 0)),
            pl.BlockSpec((D_FEAT, D_FEAT), lambda i: (0, 0)),
            pl.BlockSpec((D_FEAT, D_FEAT), lambda i: (0, 0)),
        ],
        out_specs=[
            pl.BlockSpec((_TC1_BLOCK, D_FEAT), lambda i: (i, 0)),
            pl.BlockSpec((_TC1_BLOCK, D_FEAT), lambda i: (i, 0)),
        ],
        out_shape=[
            jax.ShapeDtypeStruct((N_NODES, D_FEAT), jnp.float32),
            jax.ShapeDtypeStruct((N_NODES, D_FEAT), jnp.float32),
        ],
    )(vdata, ws_t, wr_t)


_NW = 32
_EPW = N_EDGES // _NW
_C = 128
_FULL = _EPW // _C
_TAIL = _EPW - _FULL * _C


def _sc_gather_sum(senders, receivers, ps, pr):
    mesh = plsc.VectorSubcoreMesh(core_axis_name="c", subcore_axis_name="s")

    @functools.partial(
        pl.kernel,
        mesh=mesh,
        out_type=jax.ShapeDtypeStruct((N_EDGES, D_FEAT), jnp.float32),
        scratch_types=[
            pltpu.VMEM((_EPW,), jnp.int32),
            pltpu.VMEM((_EPW,), jnp.int32),
            pltpu.VMEM((_C, D_FEAT), jnp.float32),
            pltpu.VMEM((_C, D_FEAT), jnp.float32),
            pltpu.SemaphoreType.DMA,
            pltpu.SemaphoreType.DMA,
            pltpu.SemaphoreType.DMA,
            pltpu.SemaphoreType.DMA,
        ],
    )
    def k(sidx_hbm, ridx_hbm, ps_hbm, pr_hbm, out_hbm,
          sidx_v, ridx_v, rows0, rows1, semg0, semg1, semw0, semw1):
        wid = lax.axis_index("s") * 2 + lax.axis_index("c")
        base = wid * _EPW
        pltpu.sync_copy(sidx_hbm.at[pl.ds(base, _EPW)], sidx_v)
        pltpu.sync_copy(ridx_hbm.at[pl.ds(base, _EPW)], ridx_v)

        slots = ((rows0, semg0, semw0), (rows1, semg1, semw1))

        def gs(ci, rows, semg):
            pltpu.async_copy(ps_hbm.at[sidx_v.at[pl.ds(ci * _C, _C)]],
                             rows, semg)

        def ga(ci, rows, semg):
            pltpu.async_copy(pr_hbm.at[ridx_v.at[pl.ds(ci * _C, _C)]],
                             rows, semg, add=True)

        def wait_gather(rows, semg):
            pltpu.make_async_copy(ps_hbm.at[pl.ds(0, _C)], rows, semg).wait()

        def wait_write(rows, semw):
            pltpu.make_async_copy(rows, out_hbm.at[pl.ds(0, _C)], semw).wait()

        gs(0, rows0, semg0)
        gs(1, rows1, semg1)

        def pair_body(j, carry):
            for b, (rows, semg, semw) in enumerate(slots):
                ci = 2 * j + b
                wait_gather(rows, semg)
                ga(ci, rows, semg)
                wait_gather(rows, semg)
                pltpu.async_copy(rows, out_hbm.at[pl.ds(base + ci * _C, _C)],
                                 semw)

                @pl.when(ci + 2 < _FULL)
                def _():
                    wait_write(rows, semw)
                    gs(ci + 2, rows, semg)

            return carry

        lax.fori_loop(0, _FULL // 2, pair_body, 0)

        wait_write(rows0, semw0)
        wait_write(rows1, semw1)

        toff = _FULL * _C
        rows_t = rows0.at[pl.ds(0, _TAIL)]
        pltpu.async_copy(
            ps_hbm.at[sidx_v.at[pl.ds(toff, _TAIL)]], rows_t, semg0).wait()
        pltpu.async_copy(
            pr_hbm.at[ridx_v.at[pl.ds(toff, _TAIL)]], rows_t, semg0,
            add=True).wait()
        pltpu.sync_copy(rows_t, out_hbm.at[pl.ds(base + toff, _TAIL)])

    return k(senders, receivers, ps, pr)


_TC2_BLOCK = 4000


def _tc2_body(g_ref, ed_ref, we_ref, b_ref, out_ref):
    prod = jnp.dot(ed_ref[...], we_ref[...], preferred_element_type=jnp.float32)
    out_ref[...] = g_ref[...] + prod + b_ref[...]


def _edge_update(gathered, edata, we_t, b2d):
    grid = N_EDGES // _TC2_BLOCK
    return pl.pallas_call(
        _tc2_body,
        grid=(grid,),
        in_specs=[
            pl.BlockSpec((_TC2_BLOCK, D_FEAT), lambda i: (i, 0)),
            pl.BlockSpec((_TC2_BLOCK, D_EDGE), lambda i: (i, 0)),
            pl.BlockSpec((D_EDGE, D_FEAT), lambda i: (0, 0)),
            pl.BlockSpec((1, D_FEAT), lambda i: (0, 0)),
        ],
        out_specs=pl.BlockSpec((_TC2_BLOCK, D_FEAT), lambda i: (i, 0)),
        out_shape=jax.ShapeDtypeStruct((N_EDGES, D_FEAT), jnp.float32),
    )(gathered, edata, we_t, b2d)


def kernel(vdata, edata, connectivity, W, b):
    senders = connectivity[0].astype(jnp.int32)
    receivers = connectivity[1].astype(jnp.int32)
    we_t = W[:, :D_EDGE].T
    ws_t = W[:, D_EDGE:D_EDGE + D_FEAT].T
    wr_t = W[:, D_EDGE + D_FEAT:].T
    ps, pr = _node_projections(vdata, ws_t, wr_t)
    gathered = _sc_gather_sum(senders, receivers, ps, pr)
    return _edge_update(gathered, edata, we_t, b.reshape(1, D_FEAT))
